# Initial kernel scaffold; baseline (speedup 1.0000x reference)
import jax
import jax.numpy as jnp
from jax.experimental import pallas as pl


def kernel(x, table, bias):
    raise NotImplementedError("write your pallas kernel here")


# speedup vs baseline: 2.2681x; 2.2681x over previous
import functools

import jax
import jax.numpy as jnp
from jax import lax
from jax.experimental import pallas as pl
from jax.experimental.pallas import tpu as pltpu
from jax.experimental.pallas import tpu_sc as plsc

NUM_TOKENS = 1000000
BATCH = 16384
SEQ_LEN = 50
DIM = 64

_info = plsc.get_sparse_core_info()
_NC, _NS, _L = _info.num_cores, _info.num_subcores, _info.num_lanes
_NW = _NC * _NS

_SAMPLES_PER_ROW = 2
_IDX_PER_ROW = _SAMPLES_PER_ROW * SEQ_LEN
_ROWS_PER_BLOCK = 8
_SAMPLES_PER_BLOCK = _ROWS_PER_BLOCK * _SAMPLES_PER_ROW
_SAMPLES_PER_WORKER = BATCH // _NW
_BLOCKS_PER_WORKER = _SAMPLES_PER_WORKER // _SAMPLES_PER_BLOCK
_NGROUPS = DIM // _L


def _bow_body(x_hbm, table_hbm, bias_hbm, out_hbm,
              idx_v, rows_v, out_v, bias_v, sem):
    wid = lax.axis_index("s") * _NC + lax.axis_index("c")

    pltpu.sync_copy(bias_hbm, bias_v)

    def block_body(b, _):
        row_base = wid * (_SAMPLES_PER_WORKER // _SAMPLES_PER_ROW) + b * _ROWS_PER_BLOCK
        sample_base = wid * _SAMPLES_PER_WORKER + b * _SAMPLES_PER_BLOCK

        pltpu.sync_copy(x_hbm.at[pl.ds(row_base, _ROWS_PER_BLOCK), :], idx_v)

        copies = [
            pltpu.async_copy(table_hbm.at[idx_v.at[j]], rows_v.at[j], sem)
            for j in range(_ROWS_PER_BLOCK)
        ]
        for c in copies:
            c.wait()

        def sample_body(s, _):
            j = s // _SAMPLES_PER_ROW
            off = (s % _SAMPLES_PER_ROW) * SEQ_LEN
            for g in range(_NGROUPS):
                acc = bias_v[pl.ds(g * _L, _L)]
                for r in range(SEQ_LEN):
                    acc = acc + rows_v[j, off + r, pl.ds(g * _L, _L)]
                out_v[s, pl.ds(g * _L, _L)] = acc
            return 0

        lax.fori_loop(0, _SAMPLES_PER_BLOCK, sample_body, 0)

        pltpu.sync_copy(out_v, out_hbm.at[pl.ds(sample_base, _SAMPLES_PER_BLOCK), :])
        return 0

    lax.fori_loop(0, _BLOCKS_PER_WORKER, block_body, 0)


@functools.partial(jax.jit, static_argnums=())
def _bow_call(x2, table, bias):
    mesh = plsc.VectorSubcoreMesh(core_axis_name="c", subcore_axis_name="s")
    f = functools.partial(
        pl.kernel,
        mesh=mesh,
        out_type=jax.ShapeDtypeStruct((BATCH, DIM), jnp.float32),
        scratch_types=[
            pltpu.VMEM((_ROWS_PER_BLOCK, _IDX_PER_ROW), jnp.int32),
            pltpu.VMEM((_ROWS_PER_BLOCK, _IDX_PER_ROW, DIM), jnp.float32),
            pltpu.VMEM((_SAMPLES_PER_BLOCK, DIM), jnp.float32),
            pltpu.VMEM((DIM,), jnp.float32),
            pltpu.SemaphoreType.DMA,
        ],
        compiler_params=pltpu.CompilerParams(use_tc_tiling_on_sc=False),
    )(_bow_body)
    return f(x2, table, bias)


def kernel(x, table, bias):
    x2 = x.reshape(BATCH // _SAMPLES_PER_ROW, _IDX_PER_ROW).astype(jnp.int32)
    logits = _bow_call(x2, table, bias)
    return (logits[:, :16], logits[:, 16:32], logits[:, 32:])


# speedup vs baseline: 2.6457x; 1.1665x over previous
import functools

import jax
import jax.numpy as jnp
from jax import lax
from jax.experimental import pallas as pl
from jax.experimental.pallas import tpu as pltpu
from jax.experimental.pallas import tpu_sc as plsc

NUM_TOKENS = 1000000
BATCH = 16384
SEQ_LEN = 50
DIM = 64

_info = plsc.get_sparse_core_info()
_NC, _NS, _L = _info.num_cores, _info.num_subcores, _info.num_lanes
_NW = _NC * _NS

_SAMPLES_PER_ROW = 2
_IDX_PER_ROW = _SAMPLES_PER_ROW * SEQ_LEN
_ROWS_PER_BLOCK = 8
_SAMPLES_PER_BLOCK = _ROWS_PER_BLOCK * _SAMPLES_PER_ROW
_SAMPLES_PER_WORKER = BATCH // _NW
_BLOCKS_PER_WORKER = _SAMPLES_PER_WORKER // _SAMPLES_PER_BLOCK
_NGROUPS = DIM // _L
_X_ROWS_PER_WORKER = _SAMPLES_PER_WORKER // _SAMPLES_PER_ROW


def _bow_body(x_hbm, table_hbm, bias_hbm, out_hbm,
              idx_v, rows_v, out_v, bias_v, sem0, sem1):
    wid = lax.axis_index("s") * _NC + lax.axis_index("c")
    sems = (sem0, sem1)

    pltpu.sync_copy(bias_hbm, bias_v)

    def fire(slot, b):
        row_base = wid * _X_ROWS_PER_WORKER + b * _ROWS_PER_BLOCK
        pltpu.sync_copy(x_hbm.at[pl.ds(row_base, _ROWS_PER_BLOCK), :],
                        idx_v.at[slot])
        for j in range(_ROWS_PER_BLOCK):
            pltpu.async_copy(table_hbm.at[idx_v.at[slot, j]],
                             rows_v.at[slot, j], sems[slot])

    def drain(slot):
        for j in range(_ROWS_PER_BLOCK):
            pltpu.make_async_copy(table_hbm.at[idx_v.at[slot, j]],
                                  rows_v.at[slot, j], sems[slot]).wait()

    def compute(slot, b):
        sample_base = wid * _SAMPLES_PER_WORKER + b * _SAMPLES_PER_BLOCK

        def sample_body(s, _):
            j = s // _SAMPLES_PER_ROW
            off = (s % _SAMPLES_PER_ROW) * SEQ_LEN
            for g in range(_NGROUPS):
                acc_a = bias_v[pl.ds(g * _L, _L)]
                acc_b = rows_v[slot, j, off, pl.ds(g * _L, _L)]
                for r in range(1, SEQ_LEN, 2):
                    acc_a = acc_a + rows_v[slot, j, off + r, pl.ds(g * _L, _L)]
                    if r + 1 < SEQ_LEN:
                        acc_b = acc_b + rows_v[slot, j, off + r + 1, pl.ds(g * _L, _L)]
                out_v[s, pl.ds(g * _L, _L)] = acc_a + acc_b
            return 0

        lax.fori_loop(0, _SAMPLES_PER_BLOCK, sample_body, 0)
        pltpu.sync_copy(out_v,
                        out_hbm.at[pl.ds(sample_base, _SAMPLES_PER_BLOCK), :])

    fire(0, 0)
    def pair_body(i, _):
        for phase in range(2):
            b = 2 * i + phase
            cur, nxt = phase, 1 - phase

            @pl.when(b + 1 < _BLOCKS_PER_WORKER)
            def _():
                fire(nxt, b + 1)

            drain(cur)
            compute(cur, b)
        return 0

    lax.fori_loop(0, _BLOCKS_PER_WORKER // 2, pair_body, 0)


@functools.partial(jax.jit, static_argnums=())
def _bow_call(x2, table, bias):
    mesh = plsc.VectorSubcoreMesh(core_axis_name="c", subcore_axis_name="s")
    f = functools.partial(
        pl.kernel,
        mesh=mesh,
        out_type=jax.ShapeDtypeStruct((BATCH, DIM), jnp.float32),
        scratch_types=[
            pltpu.VMEM((2, _ROWS_PER_BLOCK, _IDX_PER_ROW), jnp.int32),
            pltpu.VMEM((2, _ROWS_PER_BLOCK, _IDX_PER_ROW, DIM), jnp.float32),
            pltpu.VMEM((_SAMPLES_PER_BLOCK, DIM), jnp.float32),
            pltpu.VMEM((DIM,), jnp.float32),
            pltpu.SemaphoreType.DMA,
            pltpu.SemaphoreType.DMA,
        ],
        compiler_params=pltpu.CompilerParams(use_tc_tiling_on_sc=False),
    )(_bow_body)
    return f(x2, table, bias)


def kernel(x, table, bias):
    x2 = x.reshape(BATCH // _SAMPLES_PER_ROW, _IDX_PER_ROW).astype(jnp.int32)
    logits = _bow_call(x2, table, bias)
    return (logits[:, :16], logits[:, 16:32], logits[:, 32:])
